# baseline (speedup 1.0000x reference)
import jax
import jax.numpy as jnp
from jax.experimental import pallas as pl
from jax.experimental.pallas import tpu as pltpu

_N_STATES = 16
_HEAD = 128


def _gibbs_body(x_any, w16_ref, g_ref, o_any, xh_scr, head_scr,
                sem_b, sem_x, sem_h):
    n_chains = g_ref.shape[0]
    bulk = pltpu.make_async_copy(
        x_any.at[:, _HEAD:], o_any.at[:, _HEAD:], sem_b)
    bulk.start()
    xh_dma = pltpu.make_async_copy(x_any.at[:, :_HEAD], xh_scr, sem_x)
    xh_dma.start()
    logits = w16_ref[...] + g_ref[...]
    m = jnp.max(logits, axis=1, keepdims=True)
    iota_s = jax.lax.broadcasted_iota(jnp.int32, (n_chains, _N_STATES), 1)
    sel = jnp.min(jnp.where(logits == m, iota_s, _N_STATES), axis=1,
                  keepdims=True)
    lane = jax.lax.broadcasted_iota(jnp.int32, (n_chains, _HEAD), 1)
    onehot = (lane == sel).astype(g_ref.dtype)
    xh_dma.wait()
    head_scr[...] = jnp.where(lane < _N_STATES, onehot, xh_scr[...])
    head = pltpu.make_async_copy(head_scr, o_any.at[:, :_HEAD], sem_h)
    head.start()
    bulk.wait()
    head.wait()


def kernel(x, W):
    n_chains, n_dims, n_states = x.shape
    flat = n_dims * n_states
    x2 = x.reshape(n_chains, flat)
    w16 = W[:n_states].reshape(1, n_states)
    g = jax.random.gumbel(jax.random.key(42), (n_chains, n_states),
                          dtype=x.dtype)
    out = pl.pallas_call(
        _gibbs_body,
        in_specs=[
            pl.BlockSpec(memory_space=pltpu.MemorySpace.HBM),
            pl.BlockSpec(memory_space=pltpu.MemorySpace.VMEM),
            pl.BlockSpec(memory_space=pltpu.MemorySpace.VMEM),
        ],
        out_specs=pl.BlockSpec(memory_space=pltpu.MemorySpace.HBM),
        out_shape=jax.ShapeDtypeStruct((n_chains, flat), x.dtype),
        scratch_shapes=[
            pltpu.VMEM((n_chains, _HEAD), x.dtype),
            pltpu.VMEM((n_chains, _HEAD), x.dtype),
            pltpu.SemaphoreType.DMA,
            pltpu.SemaphoreType.DMA,
            pltpu.SemaphoreType.DMA,
        ],
    )(x2, w16, g)
    return out.reshape(n_chains, n_dims, n_states)
